# baseline (speedup 1.0000x reference)
import functools

import jax
import jax.numpy as jnp
from jax import lax
from jax.experimental import pallas as pl
from jax.experimental.pallas import tpu as pltpu
from jax.experimental.pallas import tpu_sc as plsc

N = 10000
D = 128
DLAT = 64
E = 320000

NC = 2
NS = 16
NW = NC * NS
CHUNK = 128
NCHUNK = 80
EPW = NCHUNK * CHUNK
EPAD = EPW * NW
FAST_CORE = 0
NCA = 100
NCB = 57
SSK = 2
EPAD2 = (NCA + NCB) * NS * CHUNK
NACC = 10112
ROWS_PER_TILE = NACC // NS
DEGW = 128

_MESH = plsc.VectorSubcoreMesh(core_axis_name="c", subcore_axis_name="s")


def _segsum_sc(feat, srcp, dstp, zeros):
    @functools.partial(
        pl.kernel,
        mesh=_MESH,
        out_type=jax.ShapeDtypeStruct((NC, NACC, D), jnp.float32),
        scratch_types=(
            [pltpu.VMEM((CHUNK,), jnp.int32)] * (2 * SSK)
            + [pltpu.VMEM((CHUNK, D), jnp.float32)] * SSK
            + [pltpu.VMEM_SHARED((NACC, D), jnp.float32)]
            + [pltpu.SemaphoreType.DMA] * (SSK + 2)
        ),
    )
    def k(feat_hbm, src_hbm, dst_hbm, zero_hbm, out_hbm, *rest):
        srcs = rest[:SSK]
        dsts = rest[SSK:2 * SSK]
        bufs = rest[2 * SSK:3 * SSK]
        acc = rest[3 * SSK]
        gsem = rest[3 * SSK + 1:3 * SSK + 1 + SSK]
        isem, ssem = rest[3 * SSK + 1 + SSK:]
        c = lax.axis_index("c")
        s = lax.axis_index("s")
        r0 = s * ROWS_PER_TILE
        pltpu.sync_copy(zero_hbm.at[pl.ds(r0, ROWS_PER_TILE)],
                        acc.at[pl.ds(r0, ROWS_PER_TILE)])
        plsc.subcore_barrier()

        @pl.when(c == FAST_CORE)
        def _():
            base = s * NCA * CHUNK

            def bodyf(g, carry):
                off = base + g * SSK * CHUNK
                icp = ([pltpu.async_copy(
                            src_hbm.at[pl.ds(off + b * CHUNK, CHUNK)],
                            srcs[b], isem) for b in range(SSK)]
                       + [pltpu.async_copy(
                            dst_hbm.at[pl.ds(off + b * CHUNK, CHUNK)],
                            dsts[b], isem) for b in range(SSK)])
                for cp in icp:
                    cp.wait()
                gcp = [pltpu.async_copy(feat_hbm.at[srcs[b]], bufs[b],
                                        gsem[b]) for b in range(SSK)]
                scp = []
                for b in range(SSK):
                    gcp[b].wait()
                    scp.append(pltpu.async_copy(bufs[b], acc.at[dsts[b]],
                                                ssem, add=True))
                for cp in scp:
                    cp.wait()
                return carry

            lax.fori_loop(0, NCA // SSK, bodyf, 0)

        @pl.when(c != FAST_CORE)
        def _():
            base = (NS * NCA + s * NCB) * CHUNK

            def bodys(i, carry):
                off = base + i * CHUNK
                pltpu.sync_copy(src_hbm.at[pl.ds(off, CHUNK)], srcs[0])
                pltpu.sync_copy(dst_hbm.at[pl.ds(off, CHUNK)], dsts[0])
                pltpu.async_copy(feat_hbm.at[srcs[0]], bufs[0],
                                 gsem[0]).wait()
                pltpu.sync_copy(bufs[0], acc.at[dsts[0]], add=True)
                return carry

            lax.fori_loop(0, NCB, bodys, 0)
        plsc.subcore_barrier()
        pltpu.sync_copy(acc.at[pl.ds(r0, ROWS_PER_TILE)],
                        out_hbm.at[c, pl.ds(r0, ROWS_PER_TILE)])

    return k(feat, srcp, dstp, zeros)


_DEGK = 8


def _degcount_sc(dst3, ones, zeros16):
    @functools.partial(
        pl.kernel,
        mesh=_MESH,
        out_type=jax.ShapeDtypeStruct((NC, NACC, DEGW), jnp.float32),
        scratch_types=(
            [pltpu.VMEM((CHUNK,), jnp.int32)] * _DEGK
            + [pltpu.VMEM((CHUNK, DEGW), jnp.float32),
               pltpu.VMEM_SHARED((NACC, DEGW), jnp.float32),
               pltpu.SemaphoreType.DMA,
               pltpu.SemaphoreType.DMA]
        ),
    )
    def k(dst_hbm, ones_hbm, zero_hbm, out_hbm, *rest):
        idxs = rest[:_DEGK]
        ones_v, acc, isem, sem = rest[_DEGK:]
        c = lax.axis_index("c")
        s = lax.axis_index("s")
        wid = s * NC + c
        r0 = s * ROWS_PER_TILE
        pltpu.sync_copy(zero_hbm.at[pl.ds(r0, ROWS_PER_TILE)],
                        acc.at[pl.ds(r0, ROWS_PER_TILE)])
        pltpu.sync_copy(ones_hbm, ones_v)
        plsc.subcore_barrier()
        base = wid * NCHUNK

        def body(g, carry):
            off = (base + g * _DEGK) * CHUNK
            icp = [pltpu.async_copy(
                dst_hbm.at[pl.ds(off + b * CHUNK, CHUNK)],
                idxs[b], isem) for b in range(_DEGK)]
            for cp in icp:
                cp.wait()
            scp = [pltpu.async_copy(ones_v, acc.at[idxs[b]],
                                    sem, add=True) for b in range(_DEGK)]
            for cp in scp:
                cp.wait()
            return carry

        lax.fori_loop(0, NCHUNK // _DEGK, body, 0)
        plsc.subcore_barrier()
        pltpu.sync_copy(acc.at[pl.ds(r0, ROWS_PER_TILE)],
                        out_hbm.at[c, pl.ds(r0, ROWS_PER_TILE)])

    return k(dst3, ones, zeros16)


def _prep_tc(deg_parts, x):
    def body(dp_ref, x_ref, dinv_ref, xt_ref):
        deg = dp_ref[0, :N, 0:1] + dp_ref[1, :N, 0:1] + 1.0
        dinv = lax.rsqrt(deg)
        dinv_ref[...] = jnp.broadcast_to(dinv, (N, 8))
        xt_ref[:N, :] = x_ref[...] * dinv
        xt_ref[N:, :] = jnp.zeros((NACC - N, D), jnp.float32)

    return pl.pallas_call(
        body,
        out_shape=(jax.ShapeDtypeStruct((N, 8), jnp.float32),
                   jax.ShapeDtypeStruct((NACC, D), jnp.float32)),
    )(deg_parts, x)


def _hidden_tc(P, xt, dinv8, W1):
    def body(p_ref, xt_ref, dinv_ref, w1_ref, ht_ref):
        dinv = dinv_ref[:, 0:1]
        a1 = dinv * (p_ref[0, :N, :] + p_ref[1, :N, :] + xt_ref[:N, :])
        h = jnp.maximum(
            jnp.dot(a1, w1_ref[...], preferred_element_type=jnp.float32), 0.0)
        ht_ref[:N, :] = h * dinv
        ht_ref[N:, :] = jnp.zeros((NACC - N, D), jnp.float32)

    return pl.pallas_call(
        body,
        out_shape=jax.ShapeDtypeStruct((NACC, D), jnp.float32),
    )(P, xt, dinv8, W1)


def _heads_tc(P2, ht, dinv8, Wmu, Wls, Wdec, bdec2d, mask):
    def body(p_ref, ht_ref, dinv_ref, wmu_ref, wls_ref, wd_ref, b_ref,
             m_ref, mu_ref, ls_ref, ex_ref):
        dinv = dinv_ref[:, 0:1]
        a2 = dinv * (p_ref[0, :N, :] + p_ref[1, :N, :] + ht_ref[:N, :])
        mu = jnp.dot(a2, wmu_ref[...], preferred_element_type=jnp.float32)
        mu_ref[...] = mu
        ls_ref[...] = jnp.dot(a2, wls_ref[...],
                              preferred_element_type=jnp.float32)
        wd = wd_ref[...] * m_ref[...]
        logits = lax.dot_general(mu, wd, (((1,), (1,)), ((), ())),
                                 preferred_element_type=jnp.float32)
        logits = logits + b_ref[...]
        mx = jnp.max(logits, axis=1, keepdims=True)
        e = jnp.exp(logits - mx)
        ex_ref[...] = e / jnp.sum(e, axis=1, keepdims=True)

    return pl.pallas_call(
        body,
        out_shape=(jax.ShapeDtypeStruct((N, DLAT), jnp.float32),
                   jax.ShapeDtypeStruct((N, DLAT), jnp.float32),
                   jax.ShapeDtypeStruct((N, D), jnp.float32)),
    )(P2, ht, dinv8, Wmu, Wls, Wdec, bdec2d, mask)


_TM, _TN = 1024, 2048


def _adj_tc(mu):
    def body(a_ref, b_ref, o_ref):
        o_ref[...] = lax.dot_general(
            a_ref[...], b_ref[...], (((1,), (1,)), ((), ())),
            preferred_element_type=jnp.float32)

    return pl.pallas_call(
        body,
        grid=(pl.cdiv(N, _TM), pl.cdiv(N, _TN)),
        in_specs=[pl.BlockSpec((_TM, DLAT), lambda i, j: (i, 0)),
                  pl.BlockSpec((_TN, DLAT), lambda i, j: (j, 0))],
        out_specs=pl.BlockSpec((_TM, _TN), lambda i, j: (i, j)),
        out_shape=jax.ShapeDtypeStruct((N, N), jnp.float32),
    )(mu, mu)


def kernel(x, edge_index, W1, Wmu, Wlogstd, Wdec, bdec, mask):
    src = edge_index[0].astype(jnp.int32)
    dst = edge_index[1].astype(jnp.int32)
    pad = jnp.full((EPAD - E,), N, jnp.int32)
    srcp1 = jnp.concatenate([src, pad])
    dstp1 = jnp.concatenate([dst, pad])
    pad2 = jnp.full((EPAD2 - E,), N, jnp.int32)
    srcp2 = jnp.concatenate([src, pad2])
    dstp2 = jnp.concatenate([dst, pad2])
    zeros = jnp.zeros((NACC, D), jnp.float32)
    zeros16 = jnp.zeros((NACC, DEGW), jnp.float32)
    ones16 = jnp.ones((CHUNK, DEGW), jnp.float32)

    degp = _degcount_sc(dstp1, ones16, zeros16)
    dinv8, xt = _prep_tc(degp, x)
    P1 = _segsum_sc(xt, srcp2, dstp2, zeros)
    ht = _hidden_tc(P1, xt, dinv8, W1)
    P2 = _segsum_sc(ht, srcp2, dstp2, zeros)
    mu, logstd, expr = _heads_tc(P2, ht, dinv8, Wmu, Wlogstd, Wdec,
                                 bdec.reshape(1, D), mask)
    adj = _adj_tc(mu)
    return (adj, expr, mu, logstd)


# speedup vs baseline: 1.0380x; 1.0380x over previous
import functools

import jax
import jax.numpy as jnp
from jax import lax
from jax.experimental import pallas as pl
from jax.experimental.pallas import tpu as pltpu
from jax.experimental.pallas import tpu_sc as plsc

N = 10000
D = 128
DLAT = 64
E = 320000

NC = 2
NS = 16
NW = NC * NS
CHUNK = 128
NCHUNK = 80
EPW = NCHUNK * CHUNK
EPAD = EPW * NW
FAST_CORE = 0
NCA = 106
NCB = 51
SSK = 2
EPAD2 = (NCA + NCB) * NS * CHUNK
NACC = 10112
ROWS_PER_TILE = NACC // NS
DEGW = 128

_MESH = plsc.VectorSubcoreMesh(core_axis_name="c", subcore_axis_name="s")


def _segsum_sc(feat, srcp, dstp, zeros):
    @functools.partial(
        pl.kernel,
        mesh=_MESH,
        out_type=jax.ShapeDtypeStruct((NC, NACC, D), jnp.float32),
        scratch_types=(
            [pltpu.VMEM((CHUNK,), jnp.int32)] * (2 * SSK)
            + [pltpu.VMEM((CHUNK, D), jnp.float32)] * SSK
            + [pltpu.VMEM_SHARED((NACC, D), jnp.float32)]
            + [pltpu.SemaphoreType.DMA] * (SSK + 2)
        ),
    )
    def k(feat_hbm, src_hbm, dst_hbm, zero_hbm, out_hbm, *rest):
        srcs = rest[:SSK]
        dsts = rest[SSK:2 * SSK]
        bufs = rest[2 * SSK:3 * SSK]
        acc = rest[3 * SSK]
        gsem = rest[3 * SSK + 1:3 * SSK + 1 + SSK]
        isem, ssem = rest[3 * SSK + 1 + SSK:]
        c = lax.axis_index("c")
        s = lax.axis_index("s")
        r0 = s * ROWS_PER_TILE
        pltpu.sync_copy(zero_hbm.at[pl.ds(r0, ROWS_PER_TILE)],
                        acc.at[pl.ds(r0, ROWS_PER_TILE)])
        plsc.subcore_barrier()

        @pl.when(c == FAST_CORE)
        def _():
            base = s * NCA * CHUNK

            def bodyf(g, carry):
                off = base + g * SSK * CHUNK
                icp = ([pltpu.async_copy(
                            src_hbm.at[pl.ds(off + b * CHUNK, CHUNK)],
                            srcs[b], isem) for b in range(SSK)]
                       + [pltpu.async_copy(
                            dst_hbm.at[pl.ds(off + b * CHUNK, CHUNK)],
                            dsts[b], isem) for b in range(SSK)])
                for cp in icp:
                    cp.wait()
                gcp = [pltpu.async_copy(feat_hbm.at[srcs[b]], bufs[b],
                                        gsem[b]) for b in range(SSK)]
                scp = []
                for b in range(SSK):
                    gcp[b].wait()
                    scp.append(pltpu.async_copy(bufs[b], acc.at[dsts[b]],
                                                ssem, add=True))
                for cp in scp:
                    cp.wait()
                return carry

            lax.fori_loop(0, NCA // SSK, bodyf, 0)

        @pl.when(c != FAST_CORE)
        def _():
            base = (NS * NCA + s * NCB) * CHUNK

            def bodys(i, carry):
                off = base + i * CHUNK
                pltpu.sync_copy(src_hbm.at[pl.ds(off, CHUNK)], srcs[0])
                pltpu.sync_copy(dst_hbm.at[pl.ds(off, CHUNK)], dsts[0])
                pltpu.async_copy(feat_hbm.at[srcs[0]], bufs[0],
                                 gsem[0]).wait()
                pltpu.sync_copy(bufs[0], acc.at[dsts[0]], add=True)
                return carry

            lax.fori_loop(0, NCB, bodys, 0)
        plsc.subcore_barrier()
        pltpu.sync_copy(acc.at[pl.ds(r0, ROWS_PER_TILE)],
                        out_hbm.at[c, pl.ds(r0, ROWS_PER_TILE)])

    return k(feat, srcp, dstp, zeros)


_DEGK = 8


def _degcount_sc(dst3, ones, zeros16):
    @functools.partial(
        pl.kernel,
        mesh=_MESH,
        out_type=jax.ShapeDtypeStruct((NC, NACC, DEGW), jnp.float32),
        scratch_types=(
            [pltpu.VMEM((CHUNK,), jnp.int32)] * _DEGK
            + [pltpu.VMEM((CHUNK, DEGW), jnp.float32),
               pltpu.VMEM_SHARED((NACC, DEGW), jnp.float32),
               pltpu.SemaphoreType.DMA,
               pltpu.SemaphoreType.DMA]
        ),
    )
    def k(dst_hbm, ones_hbm, zero_hbm, out_hbm, *rest):
        idxs = rest[:_DEGK]
        ones_v, acc, isem, sem = rest[_DEGK:]
        c = lax.axis_index("c")
        s = lax.axis_index("s")
        wid = s * NC + c
        r0 = s * ROWS_PER_TILE
        pltpu.sync_copy(zero_hbm.at[pl.ds(r0, ROWS_PER_TILE)],
                        acc.at[pl.ds(r0, ROWS_PER_TILE)])
        pltpu.sync_copy(ones_hbm, ones_v)
        plsc.subcore_barrier()
        base = wid * NCHUNK

        def body(g, carry):
            off = (base + g * _DEGK) * CHUNK
            icp = [pltpu.async_copy(
                dst_hbm.at[pl.ds(off + b * CHUNK, CHUNK)],
                idxs[b], isem) for b in range(_DEGK)]
            for cp in icp:
                cp.wait()
            scp = [pltpu.async_copy(ones_v, acc.at[idxs[b]],
                                    sem, add=True) for b in range(_DEGK)]
            for cp in scp:
                cp.wait()
            return carry

        lax.fori_loop(0, NCHUNK // _DEGK, body, 0)
        plsc.subcore_barrier()
        pltpu.sync_copy(acc.at[pl.ds(r0, ROWS_PER_TILE)],
                        out_hbm.at[c, pl.ds(r0, ROWS_PER_TILE)])

    return k(dst3, ones, zeros16)


def _prep_tc(deg_parts, x):
    def body(dp_ref, x_ref, dinv_ref, xt_ref):
        deg = dp_ref[0, :N, 0:1] + dp_ref[1, :N, 0:1] + 1.0
        dinv = lax.rsqrt(deg)
        dinv_ref[...] = jnp.broadcast_to(dinv, (N, 8))
        xt_ref[:N, :] = x_ref[...] * dinv
        xt_ref[N:, :] = jnp.zeros((NACC - N, D), jnp.float32)

    return pl.pallas_call(
        body,
        out_shape=(jax.ShapeDtypeStruct((N, 8), jnp.float32),
                   jax.ShapeDtypeStruct((NACC, D), jnp.float32)),
    )(deg_parts, x)


def _hidden_tc(P, xt, dinv8, W1):
    def body(p_ref, xt_ref, dinv_ref, w1_ref, ht_ref):
        dinv = dinv_ref[:, 0:1]
        a1 = dinv * (p_ref[0, :N, :] + p_ref[1, :N, :] + xt_ref[:N, :])
        h = jnp.maximum(
            jnp.dot(a1, w1_ref[...], preferred_element_type=jnp.float32), 0.0)
        ht_ref[:N, :] = h * dinv
        ht_ref[N:, :] = jnp.zeros((NACC - N, D), jnp.float32)

    return pl.pallas_call(
        body,
        out_shape=jax.ShapeDtypeStruct((NACC, D), jnp.float32),
    )(P, xt, dinv8, W1)


def _heads_tc(P2, ht, dinv8, Wmu, Wls, Wdec, bdec2d, mask):
    def body(p_ref, ht_ref, dinv_ref, wmu_ref, wls_ref, wd_ref, b_ref,
             m_ref, mu_ref, ls_ref, ex_ref):
        dinv = dinv_ref[:, 0:1]
        a2 = dinv * (p_ref[0, :N, :] + p_ref[1, :N, :] + ht_ref[:N, :])
        mu = jnp.dot(a2, wmu_ref[...], preferred_element_type=jnp.float32)
        mu_ref[...] = mu
        ls_ref[...] = jnp.dot(a2, wls_ref[...],
                              preferred_element_type=jnp.float32)
        wd = wd_ref[...] * m_ref[...]
        logits = lax.dot_general(mu, wd, (((1,), (1,)), ((), ())),
                                 preferred_element_type=jnp.float32)
        logits = logits + b_ref[...]
        mx = jnp.max(logits, axis=1, keepdims=True)
        e = jnp.exp(logits - mx)
        ex_ref[...] = e / jnp.sum(e, axis=1, keepdims=True)

    return pl.pallas_call(
        body,
        out_shape=(jax.ShapeDtypeStruct((N, DLAT), jnp.float32),
                   jax.ShapeDtypeStruct((N, DLAT), jnp.float32),
                   jax.ShapeDtypeStruct((N, D), jnp.float32)),
    )(P2, ht, dinv8, Wmu, Wls, Wdec, bdec2d, mask)


_TM, _TN = 1024, 2048


def _adj_tc(mu):
    def body(a_ref, b_ref, o_ref):
        o_ref[...] = lax.dot_general(
            a_ref[...], b_ref[...], (((1,), (1,)), ((), ())),
            preferred_element_type=jnp.float32)

    return pl.pallas_call(
        body,
        grid=(pl.cdiv(N, _TM), pl.cdiv(N, _TN)),
        in_specs=[pl.BlockSpec((_TM, DLAT), lambda i, j: (i, 0)),
                  pl.BlockSpec((_TN, DLAT), lambda i, j: (j, 0))],
        out_specs=pl.BlockSpec((_TM, _TN), lambda i, j: (i, j)),
        out_shape=jax.ShapeDtypeStruct((N, N), jnp.float32),
    )(mu, mu)


def kernel(x, edge_index, W1, Wmu, Wlogstd, Wdec, bdec, mask):
    src = edge_index[0].astype(jnp.int32)
    dst = edge_index[1].astype(jnp.int32)
    pad = jnp.full((EPAD - E,), N, jnp.int32)
    srcp1 = jnp.concatenate([src, pad])
    dstp1 = jnp.concatenate([dst, pad])
    pad2 = jnp.full((EPAD2 - E,), N, jnp.int32)
    srcp2 = jnp.concatenate([src, pad2])
    dstp2 = jnp.concatenate([dst, pad2])
    zeros = jnp.zeros((NACC, D), jnp.float32)
    zeros16 = jnp.zeros((NACC, DEGW), jnp.float32)
    ones16 = jnp.ones((CHUNK, DEGW), jnp.float32)

    degp = _degcount_sc(dstp1, ones16, zeros16)
    dinv8, xt = _prep_tc(degp, x)
    P1 = _segsum_sc(xt, srcp2, dstp2, zeros)
    ht = _hidden_tc(P1, xt, dinv8, W1)
    P2 = _segsum_sc(ht, srcp2, dstp2, zeros)
    mu, logstd, expr = _heads_tc(P2, ht, dinv8, Wmu, Wlogstd, Wdec,
                                 bdec.reshape(1, D), mask)
    adj = _adj_tc(mu)
    return (adj, expr, mu, logstd)


# speedup vs baseline: 1.0380x; 1.0000x over previous
import functools

import jax
import jax.numpy as jnp
from jax import lax
from jax.experimental import pallas as pl
from jax.experimental.pallas import tpu as pltpu
from jax.experimental.pallas import tpu_sc as plsc

N = 10000
D = 128
DLAT = 64
E = 320000

NC = 2
NS = 16
NW = NC * NS
CHUNK = 128
NCHUNK = 80
EPW = NCHUNK * CHUNK
EPAD = EPW * NW
FAST_CORE = 0
NCA = 106
NCB = 51
SSK = 2
EPAD2 = (NCA + NCB) * NS * CHUNK
NACC = 10112
ROWS_PER_TILE = NACC // NS
DEGW = 128

_MESH = plsc.VectorSubcoreMesh(core_axis_name="c", subcore_axis_name="s")


def _segsum_sc(feat, srcp, dstp, zeros):
    @functools.partial(
        pl.kernel,
        mesh=_MESH,
        out_type=jax.ShapeDtypeStruct((NC, NACC, D), jnp.float32),
        scratch_types=(
            [pltpu.VMEM((CHUNK,), jnp.int32)] * (2 * SSK)
            + [pltpu.VMEM((CHUNK, D), jnp.float32)] * SSK
            + [pltpu.VMEM_SHARED((NACC, D), jnp.float32)]
            + [pltpu.SemaphoreType.DMA] * (SSK + 2)
        ),
    )
    def k(feat_hbm, src_hbm, dst_hbm, zero_hbm, out_hbm, *rest):
        srcs = rest[:SSK]
        dsts = rest[SSK:2 * SSK]
        bufs = rest[2 * SSK:3 * SSK]
        acc = rest[3 * SSK]
        gsem = rest[3 * SSK + 1:3 * SSK + 1 + SSK]
        isem, ssem = rest[3 * SSK + 1 + SSK:]
        c = lax.axis_index("c")
        s = lax.axis_index("s")
        r0 = s * ROWS_PER_TILE
        pltpu.sync_copy(zero_hbm.at[pl.ds(r0, ROWS_PER_TILE)],
                        acc.at[pl.ds(r0, ROWS_PER_TILE)])
        plsc.subcore_barrier()

        @pl.when(c == FAST_CORE)
        def _():
            base = s * NCA * CHUNK

            def bodyf(g, carry):
                off = base + g * SSK * CHUNK
                icp = ([pltpu.async_copy(
                            src_hbm.at[pl.ds(off + b * CHUNK, CHUNK)],
                            srcs[b], isem) for b in range(SSK)]
                       + [pltpu.async_copy(
                            dst_hbm.at[pl.ds(off + b * CHUNK, CHUNK)],
                            dsts[b], isem) for b in range(SSK)])
                for cp in icp:
                    cp.wait()
                gcp = [pltpu.async_copy(feat_hbm.at[srcs[b]], bufs[b],
                                        gsem[b]) for b in range(SSK)]
                scp = []
                for b in range(SSK):
                    gcp[b].wait()
                    scp.append(pltpu.async_copy(bufs[b], acc.at[dsts[b]],
                                                ssem, add=True))
                for cp in scp:
                    cp.wait()
                return carry

            lax.fori_loop(0, NCA // SSK, bodyf, 0)

        @pl.when(c != FAST_CORE)
        def _():
            base = (NS * NCA + s * NCB) * CHUNK

            def bodys(i, carry):
                off = base + i * CHUNK
                pltpu.sync_copy(src_hbm.at[pl.ds(off, CHUNK)], srcs[0])
                pltpu.sync_copy(dst_hbm.at[pl.ds(off, CHUNK)], dsts[0])
                pltpu.async_copy(feat_hbm.at[srcs[0]], bufs[0],
                                 gsem[0]).wait()
                pltpu.sync_copy(bufs[0], acc.at[dsts[0]], add=True)
                return carry

            lax.fori_loop(0, NCB, bodys, 0)
        plsc.subcore_barrier()
        pltpu.sync_copy(acc.at[pl.ds(r0, ROWS_PER_TILE)],
                        out_hbm.at[c, pl.ds(r0, ROWS_PER_TILE)])

    return k(feat, srcp, dstp, zeros)


_DEGK = 8


def _degcount_sc(dst3, ones, zeros16):
    @functools.partial(
        pl.kernel,
        mesh=_MESH,
        out_type=jax.ShapeDtypeStruct((NC, NACC, DEGW), jnp.float32),
        scratch_types=(
            [pltpu.VMEM((CHUNK,), jnp.int32)] * _DEGK
            + [pltpu.VMEM((CHUNK, DEGW), jnp.float32),
               pltpu.VMEM_SHARED((NACC, DEGW), jnp.float32),
               pltpu.SemaphoreType.DMA,
               pltpu.SemaphoreType.DMA]
        ),
    )
    def k(dst_hbm, ones_hbm, zero_hbm, out_hbm, *rest):
        idxs = rest[:_DEGK]
        ones_v, acc, isem, sem = rest[_DEGK:]
        c = lax.axis_index("c")
        s = lax.axis_index("s")
        wid = s * NC + c
        r0 = s * ROWS_PER_TILE
        pltpu.sync_copy(zero_hbm.at[pl.ds(r0, ROWS_PER_TILE)],
                        acc.at[pl.ds(r0, ROWS_PER_TILE)])
        pltpu.sync_copy(ones_hbm, ones_v)
        plsc.subcore_barrier()
        base = wid * NCHUNK

        def body(g, carry):
            off = (base + g * _DEGK) * CHUNK
            icp = [pltpu.async_copy(
                dst_hbm.at[pl.ds(off + b * CHUNK, CHUNK)],
                idxs[b], isem) for b in range(_DEGK)]
            for cp in icp:
                cp.wait()
            scp = [pltpu.async_copy(ones_v, acc.at[idxs[b]],
                                    sem, add=True) for b in range(_DEGK)]
            for cp in scp:
                cp.wait()
            return carry

        lax.fori_loop(0, NCHUNK // _DEGK, body, 0)
        plsc.subcore_barrier()
        pltpu.sync_copy(acc.at[pl.ds(r0, ROWS_PER_TILE)],
                        out_hbm.at[c, pl.ds(r0, ROWS_PER_TILE)])

    return k(dst3, ones, zeros16)


def _prep_tc(deg_parts, x):
    def body(dp_ref, x_ref, dinv_ref, xt_ref):
        deg = dp_ref[0, :N, 0:1] + dp_ref[1, :N, 0:1] + 1.0
        dinv = lax.rsqrt(deg)
        dinv_ref[...] = jnp.broadcast_to(dinv, (N, 8))
        xt_ref[:N, :] = x_ref[...] * dinv
        xt_ref[N:, :] = jnp.zeros((NACC - N, D), jnp.float32)

    return pl.pallas_call(
        body,
        out_shape=(jax.ShapeDtypeStruct((N, 8), jnp.float32),
                   jax.ShapeDtypeStruct((NACC, D), jnp.float32)),
    )(deg_parts, x)


def _hidden_tc(P, xt, dinv8, W1):
    def body(p_ref, xt_ref, dinv_ref, w1_ref, ht_ref):
        dinv = dinv_ref[:, 0:1]
        a1 = dinv * (p_ref[0, :N, :] + p_ref[1, :N, :] + xt_ref[:N, :])
        h = jnp.maximum(
            jnp.dot(a1, w1_ref[...], preferred_element_type=jnp.float32), 0.0)
        ht_ref[:N, :] = h * dinv
        ht_ref[N:, :] = jnp.zeros((NACC - N, D), jnp.float32)

    return pl.pallas_call(
        body,
        out_shape=jax.ShapeDtypeStruct((NACC, D), jnp.float32),
    )(P, xt, dinv8, W1)


def _heads_tc(P2, ht, dinv8, Wmu, Wls, Wdec, bdec2d, mask):
    def body(p_ref, ht_ref, dinv_ref, wmu_ref, wls_ref, wd_ref, b_ref,
             m_ref, mu_ref, ls_ref, ex_ref):
        dinv = dinv_ref[:, 0:1]
        a2 = dinv * (p_ref[0, :N, :] + p_ref[1, :N, :] + ht_ref[:N, :])
        mu = jnp.dot(a2, wmu_ref[...], preferred_element_type=jnp.float32)
        mu_ref[...] = mu
        ls_ref[...] = jnp.dot(a2, wls_ref[...],
                              preferred_element_type=jnp.float32)
        wd = wd_ref[...] * m_ref[...]
        logits = lax.dot_general(mu, wd, (((1,), (1,)), ((), ())),
                                 preferred_element_type=jnp.float32)
        logits = logits + b_ref[...]
        mx = jnp.max(logits, axis=1, keepdims=True)
        e = jnp.exp(logits - mx)
        ex_ref[...] = e / jnp.sum(e, axis=1, keepdims=True)

    return pl.pallas_call(
        body,
        out_shape=(jax.ShapeDtypeStruct((N, DLAT), jnp.float32),
                   jax.ShapeDtypeStruct((N, DLAT), jnp.float32),
                   jax.ShapeDtypeStruct((N, D), jnp.float32)),
    )(P2, ht, dinv8, Wmu, Wls, Wdec, bdec2d, mask)


_TM, _TN = 1024, 2048


def _adj_tc(mu):
    def body(a_ref, b_ref, o_ref):
        o_ref[...] = lax.dot_general(
            a_ref[...], b_ref[...], (((1,), (1,)), ((), ())),
            preferred_element_type=jnp.float32)

    return pl.pallas_call(
        body,
        grid=(pl.cdiv(N, _TM), pl.cdiv(N, _TN)),
        in_specs=[pl.BlockSpec((_TM, DLAT), lambda i, j: (i, 0)),
                  pl.BlockSpec((_TN, DLAT), lambda i, j: (j, 0))],
        out_specs=pl.BlockSpec((_TM, _TN), lambda i, j: (i, j)),
        out_shape=jax.ShapeDtypeStruct((N, N), jnp.float32),
    )(mu, mu)


def kernel(x, edge_index, W1, Wmu, Wlogstd, Wdec, bdec, mask):
    src = edge_index[0].astype(jnp.int32)
    dst = edge_index[1].astype(jnp.int32)
    pad = jnp.full((EPAD - E,), N, jnp.int32)
    dstp1 = jnp.concatenate([dst, pad])
    pad2 = jnp.full((EPAD2 - E,), N, jnp.int32)
    srcp2 = jnp.concatenate([src, pad2])
    dstp2 = jnp.concatenate([dst, pad2])
    zeros = jnp.zeros((NACC, D), jnp.float32)
    zeros16 = jnp.zeros((NACC, DEGW), jnp.float32)
    ones16 = jnp.ones((CHUNK, DEGW), jnp.float32)

    degp = _degcount_sc(dstp1, ones16, zeros16)
    dinv8, xt = _prep_tc(degp, x)
    P1 = _segsum_sc(xt, srcp2, dstp2, zeros)
    ht = _hidden_tc(P1, xt, dinv8, W1)
    P2 = _segsum_sc(ht, srcp2, dstp2, zeros)
    mu, logstd, expr = _heads_tc(P2, ht, dinv8, Wmu, Wlogstd, Wdec,
                                 bdec.reshape(1, D), mask)
    adj = _adj_tc(mu)
    return (adj, expr, mu, logstd)
